# baseline (speedup 1.0000x reference)
import functools

import jax
import jax.numpy as jnp
from jax import lax
from jax.experimental import pallas as pl
from jax.experimental.pallas import tpu as pltpu
from jax.experimental.pallas import tpu_sc as plsc

EMBED_DIM = 32
LANES = 16
PAD_DIM = 128
CHUNK = 128
NUM_CORES = 2
NUM_SUBCORES = 16
NUM_WORKERS = NUM_CORES * NUM_SUBCORES
PAD_BLOCK = 8000
RS_BLOCK = 64


def _pad_body(x_ref, o_ref):
    o_ref[:, :EMBED_DIM] = x_ref[...]


def _reshape_body(x_ref, o_ref):
    o_ref[...] = x_ref[...].reshape(o_ref.shape)


def kernel(input, table):
    batch, hist = input.shape
    num_indices = batch * hist
    b_per_w = num_indices // NUM_WORKERS
    nchunks = b_per_w // CHUNK
    assert nchunks % 2 == 0 and nchunks >= 4
    indices = input.reshape(num_indices)
    vocab = table.shape[0]

    table_pad = pl.pallas_call(
        _pad_body,
        grid=(vocab // PAD_BLOCK,),
        in_specs=[pl.BlockSpec((PAD_BLOCK, EMBED_DIM), lambda i: (i, 0))],
        out_specs=pl.BlockSpec((PAD_BLOCK, PAD_DIM), lambda i: (i, 0)),
        out_shape=jax.ShapeDtypeStruct((vocab, PAD_DIM), jnp.float32),
    )(table)

    mesh = plsc.VectorSubcoreMesh(core_axis_name="core",
                                  subcore_axis_name="subcore")

    @pl.kernel(
        out_type=jax.ShapeDtypeStruct((num_indices * EMBED_DIM,),
                                      jnp.float32),
        mesh=mesh,
        scratch_types=[
            pltpu.VMEM((b_per_w,), jnp.int32),
            pltpu.VMEM((CHUNK, PAD_DIM), jnp.float32),
            pltpu.VMEM((CHUNK, PAD_DIM), jnp.float32),
            pltpu.VMEM((CHUNK * EMBED_DIM,), jnp.float32),
            pltpu.SemaphoreType.DMA,
            pltpu.SemaphoreType.DMA,
        ],
    )
    def gather_kernel(tab_hbm, idx_hbm, out_hbm, idx_v, rows0, rows1,
                      compact, sem0, sem1):
        wid = lax.axis_index("subcore") * NUM_CORES + lax.axis_index("core")
        base = wid * b_per_w
        pltpu.sync_copy(idx_hbm.at[pl.ds(base, b_per_w)], idx_v)

        def gather(chunk, rows, sem):
            pltpu.async_copy(
                tab_hbm.at[idx_v.at[pl.ds(chunk * CHUNK, CHUNK)]], rows, sem)

        def wait_gather(chunk, rows, sem):
            pltpu.make_async_copy(
                tab_hbm.at[idx_v.at[pl.ds(chunk * CHUNK, CHUNK)]], rows,
                sem).wait()

        def extract_and_write(chunk, rows):
            @pl.loop(0, CHUNK)
            def _(r):
                row = rows.at[r]
                compact[pl.ds(r * EMBED_DIM, LANES)] = row[pl.ds(0, LANES)]
                compact[pl.ds(r * EMBED_DIM + LANES, LANES)] = (
                    row[pl.ds(LANES, LANES)])

            pltpu.sync_copy(
                compact,
                out_hbm.at[pl.ds((base + chunk * CHUNK) * EMBED_DIM,
                                 CHUNK * EMBED_DIM)])

        gather(0, rows0, sem0)
        gather(1, rows1, sem1)

        @pl.loop(0, nchunks - 2, step=2)
        def _(k):
            wait_gather(k, rows0, sem0)
            extract_and_write(k, rows0)
            gather(k + 2, rows0, sem0)
            wait_gather(k + 1, rows1, sem1)
            extract_and_write(k + 1, rows1)
            gather(k + 3, rows1, sem1)

        wait_gather(nchunks - 2, rows0, sem0)
        extract_and_write(nchunks - 2, rows0)
        wait_gather(nchunks - 1, rows1, sem1)
        extract_and_write(nchunks - 1, rows1)

    out_flat = gather_kernel(table_pad, indices)

    out2d = out_flat.reshape(batch, hist * EMBED_DIM)
    out = pl.pallas_call(
        _reshape_body,
        grid=(batch // RS_BLOCK,),
        in_specs=[pl.BlockSpec((RS_BLOCK, hist * EMBED_DIM),
                               lambda i: (i, 0))],
        out_specs=pl.BlockSpec((RS_BLOCK, hist, EMBED_DIM),
                               lambda i: (i, 0, 0)),
        out_shape=jax.ShapeDtypeStruct((batch, hist, EMBED_DIM),
                                       jnp.float32),
    )(out2d)
    return out


# speedup vs baseline: 1.2259x; 1.2259x over previous
import functools

import jax
import jax.numpy as jnp
from jax import lax
from jax.experimental import pallas as pl
from jax.experimental.pallas import tpu as pltpu
from jax.experimental.pallas import tpu_sc as plsc

EMBED_DIM = 32
LANES = 16
PAD_DIM = 128
CHUNK = 128
NUM_CORES = 2
NUM_SUBCORES = 16
NUM_WORKERS = NUM_CORES * NUM_SUBCORES
def kernel(input, table):
    batch, hist = input.shape
    num_indices = batch * hist
    b_per_w = num_indices // NUM_WORKERS
    nchunks = b_per_w // CHUNK
    assert nchunks % 2 == 0 and nchunks >= 4
    indices = input.reshape(num_indices)
    vocab = table.shape[0]

    table_pad = jnp.pad(table, ((0, 0), (0, PAD_DIM - EMBED_DIM)))

    mesh = plsc.VectorSubcoreMesh(core_axis_name="core",
                                  subcore_axis_name="subcore")

    n_total_chunks = num_indices // CHUNK
    row_elems = CHUNK * EMBED_DIM

    @pl.kernel(
        out_type=jax.ShapeDtypeStruct((n_total_chunks, row_elems),
                                      jnp.float32),
        mesh=mesh,
        scratch_types=[
            pltpu.VMEM((b_per_w,), jnp.int32),
            pltpu.VMEM((CHUNK, PAD_DIM), jnp.float32),
            pltpu.VMEM((CHUNK, PAD_DIM), jnp.float32),
            pltpu.VMEM((CHUNK * EMBED_DIM,), jnp.float32),
            pltpu.SemaphoreType.DMA,
            pltpu.SemaphoreType.DMA,
        ],
    )
    def gather_kernel(tab_hbm, idx_hbm, out_hbm, idx_v, rows0, rows1,
                      compact, sem0, sem1):
        wid = lax.axis_index("subcore") * NUM_CORES + lax.axis_index("core")
        base = wid * b_per_w
        pltpu.sync_copy(idx_hbm.at[pl.ds(base, b_per_w)], idx_v)

        def gather(chunk, rows, sem):
            pltpu.async_copy(
                tab_hbm.at[idx_v.at[pl.ds(chunk * CHUNK, CHUNK)]], rows, sem)

        def wait_gather(chunk, rows, sem):
            pltpu.make_async_copy(
                tab_hbm.at[idx_v.at[pl.ds(chunk * CHUNK, CHUNK)]], rows,
                sem).wait()

        def extract_and_write(chunk, rows):
            @pl.loop(0, CHUNK)
            def _(r):
                row = rows.at[r]
                compact[pl.ds(r * EMBED_DIM, LANES)] = row[pl.ds(0, LANES)]
                compact[pl.ds(r * EMBED_DIM + LANES, LANES)] = (
                    row[pl.ds(LANES, LANES)])

            pltpu.sync_copy(compact,
                            out_hbm.at[wid * nchunks + chunk])

        gather(0, rows0, sem0)
        gather(1, rows1, sem1)

        @pl.loop(0, nchunks - 2, step=2)
        def _(k):
            wait_gather(k, rows0, sem0)
            extract_and_write(k, rows0)
            gather(k + 2, rows0, sem0)
            wait_gather(k + 1, rows1, sem1)
            extract_and_write(k + 1, rows1)
            gather(k + 3, rows1, sem1)

        wait_gather(nchunks - 2, rows0, sem0)
        extract_and_write(nchunks - 2, rows0)
        wait_gather(nchunks - 1, rows1, sem1)
        extract_and_write(nchunks - 1, rows1)

    out2d = gather_kernel(table_pad, indices)
    return out2d.reshape(batch, hist, EMBED_DIM)


# speedup vs baseline: 1.5279x; 1.2463x over previous
import jax
import jax.numpy as jnp
from jax import lax
from jax.experimental import pallas as pl
from jax.experimental.pallas import tpu as pltpu
from jax.experimental.pallas import tpu_sc as plsc

EMBED_DIM = 32
PAD_DIM = 128
CHUNK = 128
NBUF = 4
NUM_CORES = 2
NUM_SUBCORES = 16
NUM_WORKERS = NUM_CORES * NUM_SUBCORES


def kernel(input, table):
    batch, hist = input.shape
    num_indices = batch * hist
    b_per_w = num_indices // NUM_WORKERS
    nchunks = b_per_w // CHUNK
    assert nchunks % NBUF == 0 and nchunks >= 2 * NBUF
    indices = input.reshape(num_indices)
    table_pad = jnp.pad(table, ((0, 0), (0, PAD_DIM - EMBED_DIM)))

    mesh = plsc.VectorSubcoreMesh(core_axis_name="core",
                                  subcore_axis_name="subcore")

    @pl.kernel(
        out_type=jax.ShapeDtypeStruct((num_indices, PAD_DIM), jnp.float32),
        mesh=mesh,
        scratch_types=[
            pltpu.VMEM((b_per_w,), jnp.int32),
            [pltpu.VMEM((CHUNK, PAD_DIM), jnp.float32)
             for _ in range(NBUF)],
            [pltpu.SemaphoreType.DMA for _ in range(NBUF)],
            [pltpu.SemaphoreType.DMA for _ in range(NBUF)],
        ],
    )
    def gather_kernel(tab_hbm, idx_hbm, out_hbm, idx_v, rows, gsems, wsems):
        wid = lax.axis_index("subcore") * NUM_CORES + lax.axis_index("core")
        base = wid * b_per_w
        pltpu.sync_copy(idx_hbm.at[pl.ds(base, b_per_w)], idx_v)

        def gather(chunk, b):
            pltpu.async_copy(
                tab_hbm.at[idx_v.at[pl.ds(chunk * CHUNK, CHUNK)]], rows[b],
                gsems[b])

        def wait_gather(chunk, b):
            pltpu.make_async_copy(
                tab_hbm.at[idx_v.at[pl.ds(chunk * CHUNK, CHUNK)]], rows[b],
                gsems[b]).wait()

        def out_slice(chunk):
            return out_hbm.at[pl.ds((base + chunk * CHUNK), CHUNK)]

        def write(chunk, b):
            pltpu.async_copy(rows[b], out_slice(chunk), wsems[b])

        def wait_write(chunk, b):
            pltpu.make_async_copy(rows[b], out_slice(chunk),
                                  wsems[b]).wait()

        for b in range(NBUF):
            gather(b, b)

        @pl.loop(0, nchunks - NBUF, step=NBUF)
        def _(k):
            for b in range(NBUF):
                c = k + b
                wait_gather(c, b)
                write(c, b)
                wait_write(c, b)
                gather(c + NBUF, b)

        for b in range(NBUF):
            c = nchunks - NBUF + b
            wait_gather(c, b)
            write(c, b)
            wait_write(c, b)

    out_pad = gather_kernel(table_pad, indices)
    return out_pad[:, :EMBED_DIM].reshape(batch, hist, EMBED_DIM)
